# baseline (speedup 1.0000x reference)
import functools

import jax
import jax.numpy as jnp
from jax import lax
from jax.experimental import pallas as pl
from jax.experimental.pallas import tpu as pltpu
from jax.experimental.pallas import tpu_sc as plsc

_NUM_CORES = 2
_NUM_SUBCORES = 16
_NUM_WORKERS = _NUM_CORES * _NUM_SUBCORES
_LANES = 16
_B0 = 128
_HT = 2


@functools.cache
def _build_gather(batch, hist, d):
    nblocks = batch // _B0
    blocks_per_w = nblocks // _NUM_WORKERS
    r_tiles = d // 8
    ntasks = hist // _HT
    blk_len = _B0 * hist
    assert nblocks == blocks_per_w * _NUM_WORKERS
    assert ntasks * _HT == hist and r_tiles * 8 == d
    assert ntasks >= 2 and ntasks % 2 == 0

    mesh = plsc.VectorSubcoreMesh(core_axis_name="c", subcore_axis_name="s")
    iota = lambda: lax.iota(jnp.int32, _LANES)

    @functools.partial(
        pl.kernel,
        out_type=jax.ShapeDtypeStruct((hist, r_tiles, nblocks, 8, _B0), jnp.float32),
        mesh=mesh,
        scratch_types=[
            pltpu.VMEM((blk_len,), jnp.int32),
            pltpu.VMEM((blk_len,), jnp.int32),
            pltpu.VMEM((2, _HT * _B0, d), jnp.float32),
            pltpu.VMEM((2, _HT, r_tiles, 8, _B0), jnp.float32),
            pltpu.SemaphoreType.DMA,
            pltpu.SemaphoreType.DMA,
            pltpu.SemaphoreType.DMA,
            pltpu.SemaphoreType.DMA,
        ],
        compiler_params=pltpu.CompilerParams(
            use_tc_tiling_on_sc=False, needs_layout_passes=False
        ),
    )
    def gather_kernel(table, idxflat, out5, blk_v, idxt_v, rows_v, tile_v,
                      gs0, gs1, ss0, ss1):
        wid = lax.axis_index("s") * _NUM_CORES + lax.axis_index("c")
        gsem = [gs0, gs1]
        ssem = [ss0, ss1]

        def fire_gather(t, b):
            h0 = t * _HT
            pltpu.async_copy(
                table.at[idxt_v.at[pl.ds(h0 * _B0, _HT * _B0)]],
                rows_v.at[b],
                gsem[b],
            )

        def wait_gather(b):
            pltpu.make_async_copy(
                table.at[idxt_v.at[pl.ds(0, _HT * _B0)]], rows_v.at[b], gsem[b]
            ).wait()

        def transpose_rows(b):
            def dbody(dd, carry):
                r = dd // 8
                q = dd - r * 8
                for hh in range(_HT):
                    for g in range(_B0 // _LANES):
                        vec = plsc.load_gather(
                            rows_v.at[b],
                            [iota() + (hh * _B0 + g * _LANES),
                             jnp.full((_LANES,), dd, jnp.int32)],
                        )
                        tile_v[b, hh, r, q, pl.ds(g * _LANES, _LANES)] = vec
                return carry

            lax.fori_loop(0, d, dbody, 0, unroll=False)

        def fire_store(t, b, blk):
            h0 = t * _HT
            for hh in range(_HT):
                for r in range(r_tiles):
                    pltpu.async_copy(
                        tile_v.at[b, hh, r], out5.at[h0 + hh, r, blk], ssem[b]
                    )

        def wait_store(b):
            for _ in range(_HT * r_tiles):
                pltpu.make_async_copy(
                    tile_v.at[b, 0, 0], out5.at[0, 0, 0], ssem[b]
                ).wait()

        def task_step(t, b):
            @pl.when(t >= 1)
            def _():
                wait_store(1 - b)

            @pl.when(t + 1 < ntasks)
            def _():
                fire_gather(t + 1, 1 - b)

            wait_gather(b)
            transpose_rows(b)

        for blkno in range(blocks_per_w):
            blk = wid * blocks_per_w + blkno
            pltpu.sync_copy(idxflat.at[pl.ds(blk * blk_len, blk_len)], blk_v)

            def build(h, carry):
                for g in range(_B0 // _LANES):
                    vec = plsc.load_gather(
                        blk_v, [(iota() + g * _LANES) * hist + h]
                    )
                    idxt_v[pl.ds(h * _B0 + g * _LANES, _LANES)] = vec
                return carry

            lax.fori_loop(0, hist, build, 0, unroll=False)

            fire_gather(0, 0)

            def pair(k, carry):
                for b in range(2):
                    t = 2 * k + b
                    task_step(t, b)
                    fire_store(t, b, blk)
                return carry

            lax.fori_loop(0, ntasks // 2, pair, 0, unroll=False)
            wait_store(1)

    return gather_kernel


def kernel(resids_positional_encoded, weight):
    batch, hist = resids_positional_encoded.shape
    _, d = weight.shape
    idx = resids_positional_encoded.reshape(-1).astype(jnp.int32)
    out5 = _build_gather(batch, hist, d)(weight, idx)
    out = jnp.transpose(out5, (2, 4, 0, 1, 3))
    return out.reshape(batch, hist, d)


# speedup vs baseline: 1.6480x; 1.6480x over previous
import functools

import jax
import jax.numpy as jnp
from jax import lax
from jax.experimental import pallas as pl
from jax.experimental.pallas import tpu as pltpu
from jax.experimental.pallas import tpu_sc as plsc

_NUM_CORES = 2
_NUM_SUBCORES = 16
_NUM_WORKERS = _NUM_CORES * _NUM_SUBCORES
_LANES = 16
_B0 = 128
_HT = 4


@functools.cache
def _build_format(v, d):
    assert d == 32
    n_tc = v // 128
    tail = v - n_tc * 128
    out_rows = v * d // 128
    mesh = plsc.VectorSubcoreMesh(core_axis_name="c", subcore_axis_name="s")
    iota = lambda: lax.iota(jnp.int32, _LANES)

    @functools.partial(
        pl.kernel,
        out_type=jax.ShapeDtypeStruct((out_rows, 128), jnp.float32),
        mesh=mesh,
        scratch_types=[
            pltpu.VMEM((2, d, 128), jnp.float32),
            pltpu.VMEM((2, 32, 128), jnp.float32),
            pltpu.SemaphoreType.DMA,
            pltpu.SemaphoreType.DMA,
            pltpu.SemaphoreType.DMA,
            pltpu.SemaphoreType.DMA,
        ],
        compiler_params=pltpu.CompilerParams(
            use_tc_tiling_on_sc=True, needs_layout_passes=False
        ),
    )
    def format_kernel(wt, tail16, out, in_v, ob_v, ls0, ls1, ss0, ss1):
        wid = lax.axis_index("s") * _NUM_CORES + lax.axis_index("c")
        lsem = [ls0, ls1]
        ssem = [ss0, ss1]

        def cidx(j):
            return wid + j * _NUM_WORKERS

        def fire_load(j, b):
            pltpu.async_copy(
                wt.at[:, pl.ds(cidx(j) * 128, 128)], in_v.at[b], lsem[b]
            )

        def wait_load(b):
            pltpu.make_async_copy(
                wt.at[:, pl.ds(0, 128)], in_v.at[b], lsem[b]
            ).wait()

        def transpose_c(b, nj, loff=0):
            @plsc.parallel_loop(0, nj, 1, unroll=8)
            def _(j):
                for g2 in range(8):
                    vq, dhi = g2 // 2, (g2 % 2) * _LANES
                    vec = plsc.load_gather(
                        in_v.at[b],
                        [iota() + dhi,
                         jnp.full((_LANES,), 0, jnp.int32) + (loff + 4 * j + vq)],
                    )
                    ob_v[b, j, pl.ds(g2 * _LANES, _LANES)] = vec

        def fire_store(j, b):
            pltpu.async_copy(
                ob_v.at[b], out.at[pl.ds(cidx(j) * 32, 32)], ssem[b]
            )

        def wait_store(b):
            pltpu.make_async_copy(ob_v.at[b], out.at[pl.ds(0, 32)], ssem[b]).wait()

        nloop = (n_tc + _NUM_WORKERS - 1) // _NUM_WORKERS
        fire_load(0, 0)

        npair = (nloop + 1) // 2

        def cbody(k, carry):
            for b in range(2):
                j = 2 * k + b

                @pl.when((j >= 2) & (cidx(j - 2) < n_tc))
                def _():
                    wait_store(b)

                @pl.when(cidx(j + 1) < n_tc)
                def _():
                    fire_load(j + 1, 1 - b)

                @pl.when(cidx(j) < n_tc)
                def _():
                    wait_load(b)
                    transpose_c(b, 32)
                    fire_store(j, b)

            return carry

        lax.fori_loop(0, npair, cbody, 0, unroll=False)
        for j in (2 * npair - 2, 2 * npair - 1):

            @pl.when(cidx(j) < n_tc)
            def _():
                wait_store(j % 2)

        if tail:
            @pl.when(wid == _NUM_WORKERS - 1)
            def _():
                trows = tail * d // 128
                pltpu.sync_copy(tail16, ob_v.at[0].at[pl.ds(0, trows)])
                pltpu.sync_copy(
                    ob_v.at[0].at[pl.ds(0, trows)],
                    out.at[pl.ds(n_tc * 32, trows)],
                )

    return format_kernel


@functools.cache
def _build_gather(batch, hist, d):
    nblocks = batch // _B0
    blocks_per_w = nblocks // _NUM_WORKERS
    r_tiles = d // 8
    th_tiles = hist // 8
    ntasks = hist // _HT
    assert nblocks == blocks_per_w * _NUM_WORKERS
    assert ntasks * _HT == hist and r_tiles * 8 == d and th_tiles * 8 == hist
    assert ntasks >= 2 and ntasks % 2 == 0 and 8 % _HT == 0

    mesh = plsc.VectorSubcoreMesh(core_axis_name="c", subcore_axis_name="s")
    iota = lambda: lax.iota(jnp.int32, _LANES)

    @functools.partial(
        pl.kernel,
        out_type=jax.ShapeDtypeStruct((hist, r_tiles, nblocks, 8, _B0), jnp.float32),
        mesh=mesh,
        scratch_types=[
            pltpu.VMEM((1, hist * _B0), jnp.int32),
            pltpu.VMEM((2, _HT * _B0, d), jnp.float32),
            pltpu.VMEM((2, _HT * d, _B0), jnp.float32),
            pltpu.SemaphoreType.DMA,
            pltpu.SemaphoreType.DMA,
            pltpu.SemaphoreType.DMA,
            pltpu.SemaphoreType.DMA,
        ],
        compiler_params=pltpu.CompilerParams(
            use_tc_tiling_on_sc=False, needs_layout_passes=False
        ),
    )
    def gather_kernel(table, idxb, out5, idxt_v, rows_v, tile_v,
                      gs0, gs1, ss0, ss1):
        wid = lax.axis_index("s") * _NUM_CORES + lax.axis_index("c")
        gsem = [gs0, gs1]
        ssem = [ss0, ss1]

        def fire_gather(t, b):
            h0 = t * _HT
            pltpu.async_copy(
                table.at[idxt_v.at[0, pl.ds(h0 * _B0, _HT * _B0)]],
                rows_v.at[b],
                gsem[b],
            )

        def wait_gather(b):
            pltpu.make_async_copy(
                table.at[idxt_v.at[0, pl.ds(0, _HT * _B0)]],
                rows_v.at[b],
                gsem[b],
            ).wait()

        def transpose_rows(b):
            @plsc.parallel_loop(0, d, 1, unroll=4)
            def _(dd):
                dcol = jnp.full((_LANES,), dd, jnp.int32)
                base = iota()
                for hh in range(_HT):
                    for g in range(_B0 // _LANES):
                        vec = plsc.load_gather(
                            rows_v.at[b],
                            [base + (hh * _B0 + g * _LANES), dcol],
                        )
                        tile_v[b, hh * d + dd, pl.ds(g * _LANES, _LANES)] = vec

        def fire_store(t, b, blk):
            h0 = t * _HT
            for hh in range(_HT):
                for r in range(r_tiles):
                    pltpu.async_copy(
                        tile_v.at[b].at[pl.ds(hh * d + r * 8, 8)],
                        out5.at[h0 + hh, r, blk],
                        ssem[b],
                    )

        def wait_store(b):
            for _ in range(_HT * r_tiles):
                pltpu.make_async_copy(
                    tile_v.at[b].at[pl.ds(0, 8)], out5.at[0, 0, 0], ssem[b]
                ).wait()

        def task_step(t, b):
            @pl.when(t >= 1)
            def _():
                wait_store(1 - b)

            @pl.when(t + 1 < ntasks)
            def _():
                fire_gather(t + 1, 1 - b)

            wait_gather(b)
            transpose_rows(b)

        for blkno in range(blocks_per_w):
            blk = wid * blocks_per_w + blkno
            def stage(th, carry):
                pltpu.sync_copy(
                    idxb.at[th, blk], idxt_v.at[0, pl.ds(th * 8 * _B0, 8 * _B0)]
                )
                return carry

            lax.fori_loop(0, th_tiles, stage, 0, unroll=False)

            fire_gather(0, 0)

            def pair(k, carry):
                for b in range(2):
                    t = 2 * k + b
                    task_step(t, b)
                    fire_store(t, b, blk)
                return carry

            lax.fori_loop(0, ntasks // 2, pair, 0, unroll=False)
            wait_store(1)

    return gather_kernel


def kernel(resids_positional_encoded, weight):
    batch, hist = resids_positional_encoded.shape
    v, d = weight.shape
    idx = resids_positional_encoded.astype(jnp.int32)
    idxb = (
        idx.reshape(batch // 128, 128, hist // 8, 8)
        .transpose(2, 0, 3, 1)
        .reshape(hist // 8, batch // 128, 8 * 128)
    )
    n_tc = v // 128
    tail16 = weight[n_tc * 128 :].reshape((v - n_tc * 128) * d // 128, 128)
    table = _build_format(v, d)(weight.T, tail16).reshape(v, d)
    out5 = _build_gather(batch, hist, d)(table, idxb)
    out = jnp.transpose(out5, (2, 4, 0, 1, 3))
    return out.reshape(batch, hist, d)
